# baseline (speedup 1.0000x reference)
import functools

import jax
import jax.numpy as jnp
from jax import lax
from jax.experimental import pallas as pl
from jax.experimental.pallas import tpu as pltpu
from jax.experimental.pallas import tpu_sc as plsc

B, L, D = 4096, 50, 32
V = 1000000
NC, NS = 2, 16
NW = NC * NS
SR = 250016
NBLK = 7812
BLK_PER_W = 246
TAIL_OFF = NBLK * 128

_mesh = plsc.VectorSubcoreMesh(
    core_axis_name="c", subcore_axis_name="s", num_cores=NC, num_subcores=NS
)


@functools.partial(
    pl.kernel,
    out_type=jax.ShapeDtypeStruct((L, D, B), jnp.float32),
    mesh=_mesh,
    scratch_types=(
        [pltpu.HBM((SR, 128), jnp.float32)]
        + [pltpu.VMEM((D, 128), jnp.float32) for _ in range(2)]
        + [pltpu.VMEM((D, 128), jnp.float32) for _ in range(2)]
        + [pltpu.VMEM((D, 64), jnp.float32), pltpu.VMEM((16, 128), jnp.float32)]
        + [pltpu.VMEM((L, 128), jnp.int32) for _ in range(3)]
        + [pltpu.VMEM((128, 128), jnp.float32) for _ in range(2)]
        + [pltpu.VMEM((D, 128), jnp.float32) for _ in range(2)]
        + [pltpu.SemaphoreType.DMA for _ in range(8)]
        + [pltpu.SemaphoreType.REGULAR]
    ),
    compiler_params=pltpu.CompilerParams(
        use_tc_tiling_on_sc=True, needs_layout_passes=False
    ),
)
def _sc_embed(idx_hbm, table_hbm, out_hbm, scratch, *rest):
    rin = rest[0:2]
    rout = rest[2:4]
    tin, tout = rest[4:6]
    idx_v, gidx, wm32 = rest[6:9]
    gbuf = rest[9:11]
    obuf = rest[11:13]
    risem = rest[13:15]
    rosem = rest[15:17]
    gsem = rest[17:19]
    osem = rest[19:21]
    xsem = rest[21]

    cid = lax.axis_index("c")
    sid = lax.axis_index("s")
    wid = sid * NC + cid
    iota = lax.iota(jnp.int32, 16)

    start = wid * 244 + jnp.minimum(wid, 4)

    def blk_off(i):
        beta = jnp.minimum(start + i, NBLK - 1)
        return pl.multiple_of(beta * 128, 128)

    def start_in(i, b):
        pltpu.async_copy(
            table_hbm.at[:, pl.ds(blk_off(i), 128)], rin[b], risem[b]
        )

    start_in(0, 0)
    start_in(1, 1)

    @pl.loop(0, BLK_PER_W, step=2)
    def _(i0):
        for b in range(2):
            i = i0 + b
            off = blk_off(i)
            pltpu.make_async_copy(
                table_hbm.at[:, pl.ds(off, 128)], rin[b], risem[b]
            ).wait()
            for e in range(8):
                rows = iota + 16 * (e % 2)
                for m in range(32):
                    cols = jnp.full((16,), 4 * m + e // 2, jnp.int32)
                    vals = plsc.load_gather(rin[b], [rows, cols])
                    rout[b][m, 16 * e:16 * (e + 1)] = vals

            srow = pl.multiple_of(lax.shift_right_logical(off, 2), 32)
            dst = scratch.at[pl.ds(srow, 32)]

            @pl.when(i >= 2)
            def _():
                pltpu.make_async_copy(rout[b], dst, rosem[b]).wait()

            pltpu.async_copy(rout[b], dst, rosem[b])

            @pl.when(i + 2 < BLK_PER_W)
            def _():
                start_in(i + 2, b)

    for b in range(2):
        off = blk_off(BLK_PER_W - 2 + b)
        srow = pl.multiple_of(lax.shift_right_logical(off, 2), 32)
        dst = scratch.at[pl.ds(srow, 32)]
        pltpu.make_async_copy(rout[b], dst, rosem[b]).wait()

    @pl.when(wid == NW - 1)
    def _():
        pltpu.sync_copy(table_hbm.at[:, pl.ds(TAIL_OFF, 64)], tin)
        for e in range(8):
            rows = iota + 16 * (e % 2)
            for m in range(16):
                cols = jnp.full((16,), 4 * m + e // 2, jnp.int32)
                vals = plsc.load_gather(tin, [rows, cols])
                tout[m, 16 * e:16 * (e + 1)] = vals
        pltpu.sync_copy(tout, scratch.at[pl.ds(TAIL_OFF // 4, 16)])

    plsc.subcore_barrier()

    @pl.when(sid == 0)
    def _():
        pl.semaphore_signal(xsem, 1, core_index=1 - cid)
        pl.semaphore_wait(xsem, 1)

    plsc.subcore_barrier()

    b0 = pl.multiple_of(wid * 128, 128)

    pltpu.sync_copy(idx_hbm.at[:, pl.ds(b0, 128)], idx_v)

    @pl.loop(0, L)
    def _(l):
        for e in range(8):
            w = idx_v[l, 16 * e:16 * (e + 1)]
            gidx[l, 16 * e:16 * (e + 1)] = lax.shift_right_logical(w, 2)
            wm32[l, 16 * e:16 * (e + 1)] = (w & 3) * 32

    def start_gather(l, b):
        pltpu.async_copy(scratch.at[gidx.at[l]], gbuf[b], gsem[b])

    start_gather(0, 0)
    start_gather(1, 1)

    @pl.loop(0, L, step=2)
    def _(l0):
        for b in range(2):
            l = l0 + b
            pltpu.make_async_copy(scratch.at[gidx.at[l]], gbuf[b], gsem[b]).wait()
            for e in range(8):
                rows = iota + 16 * e
                wrow = wm32[l, 16 * e:16 * (e + 1)]
                for d in range(32):
                    vals = plsc.load_gather(gbuf[b], [rows, wrow + d])
                    obuf[b][d, 16 * e:16 * (e + 1)] = vals

            dst = out_hbm.at[l, :, pl.ds(b0, 128)]

            @pl.when(l >= 2)
            def _():
                pltpu.make_async_copy(obuf[b], dst, osem[b]).wait()

            pltpu.async_copy(obuf[b], dst, osem[b])

            @pl.when(l + 2 < L)
            def _():
                start_gather(l + 2, b)

    for b in range(2):
        dst = out_hbm.at[L - 2 + b, :, pl.ds(b0, 128)]
        pltpu.make_async_copy(obuf[b], dst, osem[b]).wait()


def kernel(word_inputs, feature_inputs, word_seq_lengths, char_inputs,
           char_seq_lengths, char_seq_recover, word_table,
           feat_table_0, feat_table_1):
    out = _sc_embed(word_inputs.astype(jnp.int32).T, word_table.T)
    return out.transpose(2, 0, 1)
